# baseline (speedup 1.0000x reference)
import functools

import jax
import jax.numpy as jnp
from jax import lax
from jax.experimental import pallas as pl
from jax.experimental.pallas import tpu as pltpu
from jax.experimental.pallas import tpu_sc as plsc

_B, _F, _V, _D = 16384, 26, 100000, 16
_C = _F * _D
_H1, _H2 = 512, 256
_N = _B * _F
_EPS = 1e-5


def _gather_rows(flat_idx, flat_tab):
    info = plsc.get_sparse_core_info()
    nw = info.num_cores * info.num_subcores
    per_w = _N // nw
    n_chunks = 8
    chunk = per_w // n_chunks

    mesh = plsc.VectorSubcoreMesh(core_axis_name="c", subcore_axis_name="s")

    @functools.partial(
        pl.kernel,
        mesh=mesh,
        out_type=jax.ShapeDtypeStruct((_N, _D), jnp.float32),
        compiler_params=pltpu.CompilerParams(use_tc_tiling_on_sc=False),
        scratch_types=[
            pltpu.VMEM((per_w,), jnp.int32),
            pltpu.VMEM((chunk, _D), jnp.float32),
            pltpu.SemaphoreType.DMA,
        ],
    )
    def gather_kernel(idx_hbm, tab_hbm, out_hbm, idx_v, rows_v, sem):
        wid = lax.axis_index("s") * info.num_cores + lax.axis_index("c")
        base = wid * per_w
        pltpu.sync_copy(idx_hbm.at[pl.ds(base, per_w)], idx_v)

        def body(i, carry):
            off = i * chunk
            pltpu.async_copy(
                tab_hbm.at[idx_v.at[pl.ds(off, chunk)]], rows_v, sem
            ).wait()
            pltpu.sync_copy(rows_v, out_hbm.at[pl.ds(base + off, chunk)])
            return carry

        lax.fori_loop(0, n_chunks, body, 0)

    return gather_kernel(flat_idx, flat_tab)


def _stats(x):
    bb = 2048
    nb = _B // bb

    def body(x_ref, s_ref, q_ref):
        i = pl.program_id(0)
        xb = x_ref[...]
        ps = jnp.sum(xb, axis=0, keepdims=True)
        pq = jnp.sum(xb * xb, axis=0, keepdims=True)

        @pl.when(i == 0)
        def _():
            s_ref[...] = ps
            q_ref[...] = pq

        @pl.when(i != 0)
        def _():
            s_ref[...] += ps
            q_ref[...] += pq

    return pl.pallas_call(
        body,
        grid=(nb,),
        in_specs=[pl.BlockSpec((bb, _C), lambda i: (i, 0))],
        out_specs=(
            pl.BlockSpec((1, _C), lambda i: (0, 0)),
            pl.BlockSpec((1, _C), lambda i: (0, 0)),
        ),
        out_shape=(
            jax.ShapeDtypeStruct((1, _C), jnp.float32),
            jax.ShapeDtypeStruct((1, _C), jnp.float32),
        ),
    )(x)


def _mlp(x, colsum, colsq, gamma, beta, w1t, b1, w2t, b2, w3, b3):
    bb = 2048
    nb = _B // bb
    inv_b = 1.0 / _B

    def body(x_ref, s_ref, q_ref, g_ref, be_ref, w1_ref, b1_ref, w2_ref,
             b2_ref, w3_ref, b3_ref, o_ref):
        mean = s_ref[...] * inv_b
        var = q_ref[...] * inv_b - mean * mean
        scale = g_ref[...] * lax.rsqrt(var + _EPS)
        shift = be_ref[...] - mean * scale
        xn = x_ref[...] * scale + shift
        h1 = jnp.maximum(
            jnp.dot(xn, w1_ref[...], preferred_element_type=jnp.float32)
            + b1_ref[...], 0.0)
        h2 = jnp.maximum(
            jnp.dot(h1, w2_ref[...], preferred_element_type=jnp.float32)
            + b2_ref[...], 0.0)
        o = jnp.sum(h2 * w3_ref[...], axis=1, keepdims=True) + b3_ref[...]
        o_ref[...] = 1.0 / (1.0 + jnp.exp(-o))

    full = lambda i: (0, 0)
    return pl.pallas_call(
        body,
        grid=(nb,),
        in_specs=[
            pl.BlockSpec((bb, _C), lambda i: (i, 0)),
            pl.BlockSpec((1, _C), full),
            pl.BlockSpec((1, _C), full),
            pl.BlockSpec((1, _C), full),
            pl.BlockSpec((1, _C), full),
            pl.BlockSpec((_C, _H1), full),
            pl.BlockSpec((1, _H1), full),
            pl.BlockSpec((_H1, _H2), full),
            pl.BlockSpec((1, _H2), full),
            pl.BlockSpec((1, _H2), full),
            pl.BlockSpec((1, 1), full),
        ],
        out_specs=pl.BlockSpec((bb, 1), lambda i: (i, 0)),
        out_shape=jax.ShapeDtypeStruct((_B, 1), jnp.float32),
    )(x, colsum, colsq, gamma, beta, w1t, b1, w2t, b2, w3, b3)


def kernel(indices, tables, bn_gamma, bn_beta, W1, b1, W2, b2, W3, b3):
    idx = jnp.clip(indices, 0, _V - 1).astype(jnp.int32)
    flat_idx = (idx + jnp.arange(_F, dtype=jnp.int32)[None, :] * _V).reshape(-1)
    flat_tab = tables.reshape(_F * _V, _D)
    rows = _gather_rows(flat_idx, flat_tab)
    x = rows.reshape(_B, _C)
    colsum, colsq = _stats(x)
    out = _mlp(
        x, colsum, colsq,
        bn_gamma.reshape(1, _C), bn_beta.reshape(1, _C),
        W1.T, b1.reshape(1, _H1),
        W2.T, b2.reshape(1, _H2),
        W3.reshape(1, _H2), b3.reshape(1, 1),
    )
    return out.reshape(_B)


# speedup vs baseline: 4.1293x; 4.1293x over previous
import functools

import jax
import jax.numpy as jnp
from jax import lax
from jax.experimental import pallas as pl
from jax.experimental.pallas import tpu as pltpu
from jax.experimental.pallas import tpu_sc as plsc

_B, _F, _V, _D = 16384, 26, 100000, 16
_C = _F * _D
_H1, _H2 = 512, 256
_EPS = 1e-5
_BCHUNK = 8192


def _gather_xt(idx_t, tab_rows):
    info = plsc.get_sparse_core_info()
    nw = info.num_cores * info.num_subcores
    rows_per_w = _C // nw
    n_bch = _B // _BCHUNK

    mesh = plsc.VectorSubcoreMesh(core_axis_name="c", subcore_axis_name="s")

    @functools.partial(
        pl.kernel,
        mesh=mesh,
        out_type=jax.ShapeDtypeStruct((_C, _B), jnp.float32),
        compiler_params=pltpu.CompilerParams(needs_layout_passes=False),
        scratch_types=[
            pltpu.VMEM((_V,), jnp.float32),
            pltpu.VMEM((_BCHUNK,), jnp.int32),
            pltpu.VMEM((_BCHUNK,), jnp.float32),
        ],
    )
    def gather_kernel(idx_hbm, tab_hbm, out_hbm, row_v, idx_v, out_v):
        wid = lax.axis_index("s") * info.num_cores + lax.axis_index("c")

        def per_row(i, carry):
            c = wid * rows_per_w + i
            f = c // _D
            pltpu.sync_copy(tab_hbm.at[c], row_v)

            def per_bchunk(k, carry2):
                b0 = k * _BCHUNK
                pltpu.sync_copy(idx_hbm.at[f, pl.ds(b0, _BCHUNK)], idx_v)

                def per_vec(j, carry3):
                    iv = idx_v[pl.ds(j * 16, 16)]
                    out_v[pl.ds(j * 16, 16)] = plsc.load_gather(row_v, [iv])
                    return carry3

                lax.fori_loop(0, _BCHUNK // 16, per_vec, 0, unroll=8)
                pltpu.sync_copy(out_v, out_hbm.at[c, pl.ds(b0, _BCHUNK)])
                return carry2

            lax.fori_loop(0, n_bch, per_bchunk, 0)
            return carry

        lax.fori_loop(0, rows_per_w, per_row, 0)

    return gather_kernel(idx_t, tab_rows)


def _stats(xt):
    bb = 2048
    nb = _B // bb

    def body(x_ref, s_ref, q_ref):
        i = pl.program_id(0)
        xb = x_ref[...]
        ps = jnp.sum(xb, axis=1, keepdims=True)
        pq = jnp.sum(xb * xb, axis=1, keepdims=True)

        @pl.when(i == 0)
        def _():
            s_ref[...] = ps
            q_ref[...] = pq

        @pl.when(i != 0)
        def _():
            s_ref[...] += ps
            q_ref[...] += pq

    return pl.pallas_call(
        body,
        grid=(nb,),
        in_specs=[pl.BlockSpec((_C, bb), lambda i: (0, i))],
        out_specs=(
            pl.BlockSpec((_C, 1), lambda i: (0, 0)),
            pl.BlockSpec((_C, 1), lambda i: (0, 0)),
        ),
        out_shape=(
            jax.ShapeDtypeStruct((_C, 1), jnp.float32),
            jax.ShapeDtypeStruct((_C, 1), jnp.float32),
        ),
    )(xt)


def _mlp(xt, colsum, colsq, gamma, beta, w1t, b1, w2t, b2, w3, b3):
    bb = 2048
    nb = _B // bb
    inv_b = 1.0 / _B

    def body(x_ref, s_ref, q_ref, g_ref, be_ref, w1_ref, b1_ref, w2_ref,
             b2_ref, w3_ref, b3_ref, o_ref):
        mean = s_ref[...] * inv_b
        var = q_ref[...] * inv_b - mean * mean
        scale = g_ref[...] * lax.rsqrt(var + _EPS)
        shift = be_ref[...] - mean * scale
        xn_t = x_ref[...] * scale + shift
        h1 = jnp.maximum(
            lax.dot_general(xn_t, w1_ref[...], (((0,), (0,)), ((), ())),
                            preferred_element_type=jnp.float32)
            + b1_ref[...], 0.0)
        h2 = jnp.maximum(
            jnp.dot(h1, w2_ref[...], preferred_element_type=jnp.float32)
            + b2_ref[...], 0.0)
        o = jnp.sum(h2 * w3_ref[...], axis=1, keepdims=True) + b3_ref[...]
        o_ref[...] = 1.0 / (1.0 + jnp.exp(-o))

    full = lambda i: (0, 0)
    return pl.pallas_call(
        body,
        grid=(nb,),
        in_specs=[
            pl.BlockSpec((_C, bb), lambda i: (0, i)),
            pl.BlockSpec((_C, 1), full),
            pl.BlockSpec((_C, 1), full),
            pl.BlockSpec((_C, 1), full),
            pl.BlockSpec((_C, 1), full),
            pl.BlockSpec((_C, _H1), full),
            pl.BlockSpec((1, _H1), full),
            pl.BlockSpec((_H1, _H2), full),
            pl.BlockSpec((1, _H2), full),
            pl.BlockSpec((1, _H2), full),
            pl.BlockSpec((1, 1), full),
        ],
        out_specs=pl.BlockSpec((bb, 1), lambda i: (i, 0)),
        out_shape=jax.ShapeDtypeStruct((_B, 1), jnp.float32),
    )(xt, colsum, colsq, gamma, beta, w1t, b1, w2t, b2, w3, b3)


def kernel(indices, tables, bn_gamma, bn_beta, W1, b1, W2, b2, W3, b3):
    idx_t = jnp.clip(indices, 0, _V - 1).astype(jnp.int32).T
    tab_rows = tables.transpose(0, 2, 1).reshape(_C, _V)
    xt = _gather_xt(idx_t, tab_rows)
    colsum, colsq = _stats(xt)
    out = _mlp(
        xt, colsum, colsq,
        bn_gamma.reshape(_C, 1), bn_beta.reshape(_C, 1),
        W1.T, b1.reshape(1, _H1),
        W2.T, b2.reshape(1, _H2),
        W3.reshape(1, _H2), b3.reshape(1, 1),
    )
    return out.reshape(_B)


# speedup vs baseline: 6.3738x; 1.5436x over previous
import functools

import jax
import jax.numpy as jnp
from jax import lax
from jax.experimental import pallas as pl
from jax.experimental.pallas import tpu as pltpu
from jax.experimental.pallas import tpu_sc as plsc

_B, _F, _V, _D = 16384, 26, 100000, 16
_C = _F * _D
_H1, _H2 = 512, 256
_EPS = 1e-5
_BCHUNK = 8192


def _gather_xt(idx_t, tab_rows):
    info = plsc.get_sparse_core_info()
    nw = info.num_cores * info.num_subcores
    rows_per_w = _C // nw
    n_bch = _B // _BCHUNK

    mesh = plsc.VectorSubcoreMesh(core_axis_name="c", subcore_axis_name="s")

    @functools.partial(
        pl.kernel,
        mesh=mesh,
        out_type=jax.ShapeDtypeStruct((_C, _B), jnp.float32),
        compiler_params=pltpu.CompilerParams(needs_layout_passes=False),
        scratch_types=[
            pltpu.VMEM((_V,), jnp.float32),
            pltpu.VMEM((_BCHUNK,), jnp.int32),
            pltpu.VMEM((_BCHUNK,), jnp.float32),
        ],
    )
    def gather_kernel(idx_hbm, tab_hbm, out_hbm, row_v, idx_v, out_v):
        wid = lax.axis_index("s") * info.num_cores + lax.axis_index("c")

        def per_row(i, carry):
            c = wid * rows_per_w + i
            f = c // _D
            pltpu.sync_copy(tab_hbm.at[c], row_v)

            def per_bchunk(k, carry2):
                b0 = k * _BCHUNK
                pltpu.sync_copy(idx_hbm.at[f, pl.ds(b0, _BCHUNK)], idx_v)

                @plsc.parallel_loop(0, _BCHUNK // 16, unroll=8)
                def per_vec(j):
                    iv = idx_v[pl.ds(j * 16, 16)]
                    out_v[pl.ds(j * 16, 16)] = plsc.load_gather(row_v, [iv])
                pltpu.sync_copy(out_v, out_hbm.at[c, pl.ds(b0, _BCHUNK)])
                return carry2

            lax.fori_loop(0, n_bch, per_bchunk, 0)
            return carry

        lax.fori_loop(0, rows_per_w, per_row, 0)

    return gather_kernel(idx_t, tab_rows)


def _stats(xt):
    bb = 2048
    nb = _B // bb

    def body(x_ref, s_ref, q_ref):
        i = pl.program_id(0)
        xb = x_ref[...]
        ps = jnp.sum(xb, axis=1, keepdims=True)
        pq = jnp.sum(xb * xb, axis=1, keepdims=True)

        @pl.when(i == 0)
        def _():
            s_ref[...] = ps
            q_ref[...] = pq

        @pl.when(i != 0)
        def _():
            s_ref[...] += ps
            q_ref[...] += pq

    return pl.pallas_call(
        body,
        grid=(nb,),
        in_specs=[pl.BlockSpec((_C, bb), lambda i: (0, i))],
        out_specs=(
            pl.BlockSpec((_C, 1), lambda i: (0, 0)),
            pl.BlockSpec((_C, 1), lambda i: (0, 0)),
        ),
        out_shape=(
            jax.ShapeDtypeStruct((_C, 1), jnp.float32),
            jax.ShapeDtypeStruct((_C, 1), jnp.float32),
        ),
    )(xt)


def _mlp(xt, colsum, colsq, gamma, beta, w1t, b1, w2t, b2, w3, b3):
    bb = 2048
    nb = _B // bb
    inv_b = 1.0 / _B

    def body(x_ref, s_ref, q_ref, g_ref, be_ref, w1_ref, b1_ref, w2_ref,
             b2_ref, w3_ref, b3_ref, o_ref):
        mean = s_ref[...] * inv_b
        var = q_ref[...] * inv_b - mean * mean
        scale = g_ref[...] * lax.rsqrt(var + _EPS)
        shift = be_ref[...] - mean * scale
        xn_t = x_ref[...] * scale + shift
        h1 = jnp.maximum(
            lax.dot_general(xn_t, w1_ref[...], (((0,), (0,)), ((), ())),
                            preferred_element_type=jnp.float32)
            + b1_ref[...], 0.0)
        h2 = jnp.maximum(
            jnp.dot(h1, w2_ref[...], preferred_element_type=jnp.float32)
            + b2_ref[...], 0.0)
        o = jnp.sum(h2 * w3_ref[...], axis=1, keepdims=True) + b3_ref[...]
        o_ref[...] = 1.0 / (1.0 + jnp.exp(-o))

    full = lambda i: (0, 0)
    return pl.pallas_call(
        body,
        grid=(nb,),
        in_specs=[
            pl.BlockSpec((_C, bb), lambda i: (0, i)),
            pl.BlockSpec((_C, 1), full),
            pl.BlockSpec((_C, 1), full),
            pl.BlockSpec((_C, 1), full),
            pl.BlockSpec((_C, 1), full),
            pl.BlockSpec((_C, _H1), full),
            pl.BlockSpec((1, _H1), full),
            pl.BlockSpec((_H1, _H2), full),
            pl.BlockSpec((1, _H2), full),
            pl.BlockSpec((1, _H2), full),
            pl.BlockSpec((1, 1), full),
        ],
        out_specs=pl.BlockSpec((bb, 1), lambda i: (i, 0)),
        out_shape=jax.ShapeDtypeStruct((_B, 1), jnp.float32),
    )(xt, colsum, colsq, gamma, beta, w1t, b1, w2t, b2, w3, b3)


def kernel(indices, tables, bn_gamma, bn_beta, W1, b1, W2, b2, W3, b3):
    idx_t = jnp.clip(indices, 0, _V - 1).astype(jnp.int32).T
    tab_rows = tables.transpose(0, 2, 1).reshape(_C, _V)
    xt = _gather_xt(idx_t, tab_rows)
    colsum, colsq = _stats(xt)
    out = _mlp(
        xt, colsum, colsq,
        bn_gamma.reshape(_C, 1), bn_beta.reshape(_C, 1),
        W1.T, b1.reshape(1, _H1),
        W2.T, b2.reshape(1, _H2),
        W3.reshape(1, _H2), b3.reshape(1, 1),
    )
    return out.reshape(_B)
